# baseline (speedup 1.0000x reference)
import functools

import jax
import jax.numpy as jnp
from jax import lax
from jax.experimental import pallas as pl
from jax.experimental.pallas import tpu as pltpu
from jax.experimental.pallas import tpu_sc as plsc

N = 10000
E = 160000
D = 256
R = 3

NT = 32
RPT = 320
TRASH = RPT
CAP = 6144
CH = 4000
GB = 64
NGRP = CAP // GB
LAST = N - (NT - 1) * RPT
NEG = float("-inf")

_MESH = plsc.VectorSubcoreMesh(core_axis_name="c", subcore_axis_name="s")
_SC_PARAMS = pltpu.CompilerParams(needs_layout_passes=False)


@functools.partial(
    pl.kernel,
    out_type=(jax.ShapeDtypeStruct((NT, CAP), jnp.int32),
              jax.ShapeDtypeStruct((NT, CAP), jnp.int32),
              jax.ShapeDtypeStruct((NT, 16), jnp.int32)),
    mesh=_MESH,
    scratch_types=[
        pltpu.VMEM((CH,), jnp.int32),
        pltpu.VMEM((CH,), jnp.int32),
        pltpu.VMEM((CH,), jnp.int32),
        pltpu.VMEM((CH,), jnp.int32),
        pltpu.VMEM((CAP,), jnp.int32),
        pltpu.VMEM((CAP,), jnp.int32),
        pltpu.VMEM((16,), jnp.int32),
        pltpu.SemaphoreType.DMA,
        pltpu.SemaphoreType.DMA,
    ],
    compiler_params=_SC_PARAMS,
)
def _sc_filter(rows_hbm, dst_hbm, sel_rows_hbm, sel_ld_hbm, cnt_hbm,
               dstA, rowA, dstB, rowB, rows_sel, ld_sel, cntbuf, semA, semB):
  wid = lax.axis_index("s") * 2 + lax.axis_index("c")
  lo = wid * RPT
  hi = lo + RPT

  trash = TRASH + (lax.broadcasted_iota(jnp.int32, (16,), 0) & 7)

  def init_sel(j, carry):
    ld_sel[pl.ds(j * 16, 16)] = trash
    rows_sel[pl.ds(j * 16, 16)] = jnp.zeros((16,), jnp.int32)
    return carry
  lax.fori_loop(0, CAP // 16, init_sel, 0)

  def start(c, dbuf, rbuf, sem):
    pltpu.async_copy(dst_hbm.at[pl.ds(c * CH, CH)], dbuf, sem)
    pltpu.async_copy(rows_hbm.at[pl.ds(c * CH, CH)], rbuf, sem)

  def drain(dbuf, rbuf, sem):
    pltpu.make_async_copy(dst_hbm.at[pl.ds(0, CH)], dbuf, sem).wait()
    pltpu.make_async_copy(rows_hbm.at[pl.ds(0, CH)], rbuf, sem).wait()

  def process(dbuf, rbuf, cnt):
    def vreg_body(v, cnt):
      sl = pl.ds(v * 16, 16)
      d = dbuf[sl]
      r = rbuf[sl]
      m = (d >= lo) & (d < hi)
      off = jnp.minimum(cnt, CAP - 16)
      plsc.store_compressed(ld_sel.at[pl.ds(off, 16)], d - lo, mask=m)
      plsc.store_compressed(rows_sel.at[pl.ds(off, 16)], r, mask=m)
      return cnt + plsc.all_reduce_population_count(m)[0]
    return lax.fori_loop(0, CH // 16, vreg_body, cnt)

  NCH = E // CH
  start(0, dstA, rowA, semA)
  start(1, dstB, rowB, semB)

  def pair_body(p, cnt):
    drain(dstA, rowA, semA)
    cnt = process(dstA, rowA, cnt)
    @pl.when(p < NCH // 2 - 1)
    def _():
      start(2 * p + 2, dstA, rowA, semA)
    drain(dstB, rowB, semB)
    cnt = process(dstB, rowB, cnt)
    @pl.when(p < NCH // 2 - 1)
    def _():
      start(2 * p + 3, dstB, rowB, semB)
    return cnt
  cnt = lax.fori_loop(0, NCH // 2, pair_body, jnp.int32(0))

  cntbuf[pl.ds(0, 16)] = jnp.zeros((16,), jnp.int32) + cnt
  pltpu.sync_copy(rows_sel, sel_rows_hbm.at[wid])
  pltpu.sync_copy(ld_sel, sel_ld_hbm.at[wid])
  pltpu.sync_copy(cntbuf, cnt_hbm.at[wid])


def _make_sc_agg(mode):
  init_val = NEG if mode == "max" else 0.0
  combine = jnp.maximum if mode == "max" else jnp.add
  dt = jnp.bfloat16 if mode == "max" else jnp.float32
  VW = 32 if mode == "max" else 16
  GW = D // 2 if mode == "max" else D

  @functools.partial(
      pl.kernel,
      out_type=jax.ShapeDtypeStruct((N, D), dt),
      mesh=_MESH,
      scratch_types=[
          pltpu.VMEM((CAP,), jnp.int32),
          pltpu.VMEM((CAP,), jnp.int32),
          pltpu.VMEM((16,), jnp.int32),
          pltpu.VMEM((GB, GW), jnp.int32 if mode == "max" else jnp.float32),
          pltpu.VMEM((GB, GW), jnp.int32 if mode == "max" else jnp.float32),
          pltpu.VMEM((RPT + 8, D), dt),
          pltpu.SemaphoreType.DMA,
          pltpu.SemaphoreType.DMA,
      ],
      compiler_params=_SC_PARAMS,
  )
  def sc_agg(xw_hbm, sel_rows_hbm, sel_ld_hbm, cnt_hbm, out_hbm,
             rows_sel, ld_sel, cntbuf, gbufA, gbufB, acc, semA, semB):
    wid = lax.axis_index("s") * 2 + lax.axis_index("c")
    lo = wid * RPT

    pltpu.sync_copy(sel_rows_hbm.at[wid], rows_sel)
    pltpu.sync_copy(sel_ld_hbm.at[wid], ld_sel)
    pltpu.sync_copy(cnt_hbm.at[wid], cntbuf)
    cnt = cntbuf[pl.ds(0, 16)][0]

    def init_acc(j, carry):
      for k in range(D // VW):
        acc[j, pl.ds(k * VW, VW)] = jnp.full((VW,), init_val, dt)
      return carry
    lax.fori_loop(0, RPT + 8, init_acc, 0)

    def issue(g, gbuf, sem):
      pltpu.async_copy(xw_hbm.at[rows_sel.at[pl.ds(g * GB, GB)]], gbuf, sem)

    def drain(gbuf, sem):
      pltpu.make_async_copy(
          xw_hbm.at[rows_sel.at[pl.ds(0, GB)]], gbuf, sem).wait()

    def process(g, gbuf):
      def sub_body(q, c2):
        ldv = ld_sel[pl.ds(g * GB + q * 16, 16)]
        for i in range(16):
          ldi = ldv[i]
          row = q * 16 + i
          for k in range(D // VW):
            sl = pl.ds(k * VW, VW)
            mv = gbuf[row, pl.ds(k * 16, 16)]
            if mode == "max":
              mv = plsc.bitcast(mv, jnp.bfloat16)
            acc[ldi, sl] = combine(acc[ldi, sl], mv)
        return c2
      lax.fori_loop(0, GB // 16, sub_body, 0)

    ngrp = (cnt + (GB - 1)) // GB
    npairs = ngrp // 2
    @pl.when(ngrp > 0)
    def _():
      issue(0, gbufA, semA)
    @pl.when(ngrp > 1)
    def _():
      issue(1, gbufB, semB)

    def pair_body(p, carry):
      drain(gbufA, semA)
      process(2 * p, gbufA)
      @pl.when(2 * p + 2 < ngrp)
      def _():
        issue(2 * p + 2, gbufA, semA)
      drain(gbufB, semB)
      process(2 * p + 1, gbufB)
      @pl.when(2 * p + 3 < ngrp)
      def _():
        issue(2 * p + 3, gbufB, semB)
      return carry
    lax.fori_loop(0, npairs, pair_body, 0)

    @pl.when(ngrp > 2 * npairs)
    def _():
      drain(gbufA, semA)
      process(2 * npairs, gbufA)

    @pl.when(wid < NT - 1)
    def _():
      pltpu.sync_copy(acc.at[pl.ds(0, RPT)], out_hbm.at[pl.ds(lo, RPT)])

    @pl.when(wid == NT - 1)
    def _():
      pltpu.sync_copy(acc.at[pl.ds(0, LAST)], out_hbm.at[pl.ds(lo, LAST)])

  return sc_agg


_sc_max = _make_sc_agg("max")
_sc_add = _make_sc_agg("add")

BN = 1000


def _tc_layer1(x, Wrel1, Wroot1, b1):
  def body(x_ref, wr_ref, wro_ref, b_ref, xw_ref, xr_ref):
    xb = x_ref[...]
    for r in range(R):
      xw_ref[r] = jnp.dot(
          xb, wr_ref[r], preferred_element_type=jnp.float32
      ).astype(jnp.bfloat16)
    xr_ref[...] = (jnp.dot(xb, wro_ref[...], preferred_element_type=jnp.float32)
                   + b_ref[...])

  return pl.pallas_call(
      body,
      grid=(N // BN,),
      in_specs=[
          pl.BlockSpec((BN, D), lambda i: (i, 0)),
          pl.BlockSpec((R, D, D), lambda i: (0, 0, 0)),
          pl.BlockSpec((D, D), lambda i: (0, 0)),
          pl.BlockSpec((1, D), lambda i: (0, 0)),
      ],
      out_specs=[
          pl.BlockSpec((R, BN, D), lambda i: (0, i, 0)),
          pl.BlockSpec((BN, D), lambda i: (i, 0)),
      ],
      out_shape=[
          jax.ShapeDtypeStruct((R, N, D), jnp.bfloat16),
          jax.ShapeDtypeStruct((N, D), jnp.float32),
      ],
  )(x, Wrel1, Wroot1, b1.reshape(1, D))


def _layer_norm_in(h, g, b):
  mu = jnp.mean(h, axis=1, keepdims=True)
  var = jnp.mean((h - mu) ** 2, axis=1, keepdims=True)
  return (h - mu) / jnp.sqrt(var + 1e-5) * g + b


def _tc_mid(agg1, xroot1, g1, be1, Wrel2, Wroot2, b2):
  def body(a_ref, xr_ref, g_ref, be_ref, wr_ref, wro_ref, b_ref,
           xw_ref, hr_ref):
    a = a_ref[...].astype(jnp.float32)
    a = jnp.where(a == NEG, 0.0, a)
    h = a + xr_ref[...]
    h = _layer_norm_in(h, g_ref[...], be_ref[...])
    h = jnp.maximum(h, 0.0)
    for r in range(R):
      xw_ref[r] = jnp.dot(h, wr_ref[r], preferred_element_type=jnp.float32)
    hr_ref[...] = (jnp.dot(h, wro_ref[...], preferred_element_type=jnp.float32)
                   + b_ref[...])

  return pl.pallas_call(
      body,
      grid=(N // BN,),
      in_specs=[
          pl.BlockSpec((BN, D), lambda i: (i, 0)),
          pl.BlockSpec((BN, D), lambda i: (i, 0)),
          pl.BlockSpec((1, D), lambda i: (0, 0)),
          pl.BlockSpec((1, D), lambda i: (0, 0)),
          pl.BlockSpec((R, D, D), lambda i: (0, 0, 0)),
          pl.BlockSpec((D, D), lambda i: (0, 0)),
          pl.BlockSpec((1, D), lambda i: (0, 0)),
      ],
      out_specs=[
          pl.BlockSpec((R, BN, D), lambda i: (0, i, 0)),
          pl.BlockSpec((BN, D), lambda i: (i, 0)),
      ],
      out_shape=[
          jax.ShapeDtypeStruct((R, N, D), jnp.float32),
          jax.ShapeDtypeStruct((N, D), jnp.float32),
      ],
  )(agg1, xroot1, g1.reshape(1, D), be1.reshape(1, D), Wrel2, Wroot2,
    b2.reshape(1, D))


def _tc_out(agg2, hroot2, g2, be2):
  def body(a_ref, hr_ref, g_ref, be_ref, o_ref):
    z = _layer_norm_in(a_ref[...] + hr_ref[...], g_ref[...], be_ref[...])
    z = z - jnp.max(z, axis=1, keepdims=True)
    o_ref[...] = z - jnp.log(jnp.sum(jnp.exp(z), axis=1, keepdims=True))

  return pl.pallas_call(
      body,
      grid=(N // BN,),
      in_specs=[
          pl.BlockSpec((BN, D), lambda i: (i, 0)),
          pl.BlockSpec((BN, D), lambda i: (i, 0)),
          pl.BlockSpec((1, D), lambda i: (0, 0)),
          pl.BlockSpec((1, D), lambda i: (0, 0)),
      ],
      out_specs=pl.BlockSpec((BN, D), lambda i: (i, 0)),
      out_shape=jax.ShapeDtypeStruct((N, D), jnp.float32),
  )(agg2, hroot2, g2.reshape(1, D), be2.reshape(1, D))


@jax.jit
def kernel(x, edge_index, edge_type, Wrel1, Wroot1, b1, g1, be1,
           Wrel2, Wroot2, b2, g2, be2):
  src = edge_index[0]
  dst = edge_index[1]
  rows = edge_type * N + src

  sel_rows, sel_ld, cnts = _sc_filter(rows, dst)
  xw1, xroot1 = _tc_layer1(x, Wrel1, Wroot1, b1)
  xw1_i32 = lax.bitcast_convert_type(
      xw1.reshape(R * N, D // 2, 2), jnp.int32)
  agg1 = _sc_max(xw1_i32, sel_rows, sel_ld, cnts)
  xw2, hroot2 = _tc_mid(agg1, xroot1, g1, be1, Wrel2, Wroot2, b2)
  agg2 = _sc_add(xw2.reshape(R * N, D), sel_rows, sel_ld, cnts)
  return _tc_out(agg2, hroot2, g2, be2)


# speedup vs baseline: 1.0907x; 1.0907x over previous
import functools

import jax
import jax.numpy as jnp
from jax import lax
from jax.experimental import pallas as pl
from jax.experimental.pallas import tpu as pltpu
from jax.experimental.pallas import tpu_sc as plsc

N = 10000
E = 160000
D = 256
R = 3

NT = 32
RPT = 320
TRASH = RPT
CAP = 6144
CH = 4000
GB = 64
NGRP = CAP // GB
LAST = N - (NT - 1) * RPT
NEG = float("-inf")

_MESH = plsc.VectorSubcoreMesh(core_axis_name="c", subcore_axis_name="s")
_SC_PARAMS = pltpu.CompilerParams(needs_layout_passes=False)


@functools.partial(
    pl.kernel,
    out_type=(jax.ShapeDtypeStruct((NT, CAP), jnp.int32),
              jax.ShapeDtypeStruct((NT, CAP), jnp.int32),
              jax.ShapeDtypeStruct((NT, 16), jnp.int32)),
    mesh=_MESH,
    scratch_types=[
        pltpu.VMEM((CH,), jnp.int32),
        pltpu.VMEM((CH,), jnp.int32),
        pltpu.VMEM((CH,), jnp.int32),
        pltpu.VMEM((CH,), jnp.int32),
        pltpu.VMEM((CAP,), jnp.int32),
        pltpu.VMEM((CAP,), jnp.int32),
        pltpu.VMEM((16,), jnp.int32),
        pltpu.SemaphoreType.DMA,
        pltpu.SemaphoreType.DMA,
    ],
    compiler_params=_SC_PARAMS,
)
def _sc_filter(rows_hbm, dst_hbm, sel_rows_hbm, sel_ld_hbm, cnt_hbm,
               dstA, rowA, dstB, rowB, rows_sel, ld_sel, cntbuf, semA, semB):
  wid = lax.axis_index("s") * 2 + lax.axis_index("c")
  lo = wid * RPT
  hi = lo + RPT

  trash = TRASH + (lax.broadcasted_iota(jnp.int32, (16,), 0) & 7)

  def init_sel(j, carry):
    ld_sel[pl.ds(j * 16, 16)] = trash
    rows_sel[pl.ds(j * 16, 16)] = jnp.zeros((16,), jnp.int32)
    return carry
  lax.fori_loop(0, CAP // 16, init_sel, 0)

  def start(c, dbuf, rbuf, sem):
    pltpu.async_copy(dst_hbm.at[pl.ds(c * CH, CH)], dbuf, sem)
    pltpu.async_copy(rows_hbm.at[pl.ds(c * CH, CH)], rbuf, sem)

  def drain(dbuf, rbuf, sem):
    pltpu.make_async_copy(dst_hbm.at[pl.ds(0, CH)], dbuf, sem).wait()
    pltpu.make_async_copy(rows_hbm.at[pl.ds(0, CH)], rbuf, sem).wait()

  def process(dbuf, rbuf, cnt):
    def vreg_body(v, cnt):
      sl = pl.ds(v * 16, 16)
      d = dbuf[sl]
      r = rbuf[sl]
      m = (d >= lo) & (d < hi)
      off = jnp.minimum(cnt, CAP - 16)
      plsc.store_compressed(ld_sel.at[pl.ds(off, 16)], d - lo, mask=m)
      plsc.store_compressed(rows_sel.at[pl.ds(off, 16)], r, mask=m)
      return cnt + plsc.all_reduce_population_count(m)[0]
    return lax.fori_loop(0, CH // 16, vreg_body, cnt)

  NCH = E // CH
  start(0, dstA, rowA, semA)
  start(1, dstB, rowB, semB)

  def pair_body(p, cnt):
    drain(dstA, rowA, semA)
    cnt = process(dstA, rowA, cnt)
    @pl.when(p < NCH // 2 - 1)
    def _():
      start(2 * p + 2, dstA, rowA, semA)
    drain(dstB, rowB, semB)
    cnt = process(dstB, rowB, cnt)
    @pl.when(p < NCH // 2 - 1)
    def _():
      start(2 * p + 3, dstB, rowB, semB)
    return cnt
  cnt = lax.fori_loop(0, NCH // 2, pair_body, jnp.int32(0))

  cntbuf[pl.ds(0, 16)] = jnp.zeros((16,), jnp.int32) + cnt
  pltpu.sync_copy(rows_sel, sel_rows_hbm.at[wid])
  pltpu.sync_copy(ld_sel, sel_ld_hbm.at[wid])
  pltpu.sync_copy(cntbuf, cnt_hbm.at[wid])


def _make_sc_agg(mode):
  init_val = NEG if mode == "max" else 0.0
  combine = jnp.maximum if mode == "max" else jnp.add

  @functools.partial(
      pl.kernel,
      out_type=jax.ShapeDtypeStruct((N, D), jnp.float32),
      mesh=_MESH,
      scratch_types=[
          pltpu.VMEM((CAP,), jnp.int32),
          pltpu.VMEM((CAP,), jnp.int32),
          pltpu.VMEM((16,), jnp.int32),
          pltpu.VMEM((GB, D), jnp.float32),
          pltpu.VMEM((GB, D), jnp.float32),
          pltpu.VMEM((RPT + 8, D), jnp.float32),
          pltpu.SemaphoreType.DMA,
          pltpu.SemaphoreType.DMA,
      ],
      compiler_params=_SC_PARAMS,
  )
  def sc_agg(xw_hbm, sel_rows_hbm, sel_ld_hbm, cnt_hbm, out_hbm,
             rows_sel, ld_sel, cntbuf, gbufA, gbufB, acc, semA, semB):
    wid = lax.axis_index("s") * 2 + lax.axis_index("c")
    lo = wid * RPT

    pltpu.sync_copy(sel_rows_hbm.at[wid], rows_sel)
    pltpu.sync_copy(sel_ld_hbm.at[wid], ld_sel)
    pltpu.sync_copy(cnt_hbm.at[wid], cntbuf)
    cnt = cntbuf[pl.ds(0, 16)][0]

    def init_acc(j, carry):
      for k in range(D // 16):
        acc[j, pl.ds(k * 16, 16)] = jnp.full((16,), init_val, jnp.float32)
      return carry
    lax.fori_loop(0, RPT + 8, init_acc, 0)

    def issue(g, gbuf, sem):
      pltpu.async_copy(xw_hbm.at[rows_sel.at[pl.ds(g * GB, GB)]], gbuf, sem)

    def drain(gbuf, sem):
      pltpu.make_async_copy(
          xw_hbm.at[rows_sel.at[pl.ds(0, GB)]], gbuf, sem).wait()

    def process(g, gbuf):
      def sub_body(q, c2):
        ldv = ld_sel[pl.ds(g * GB + q * 16, 16)]
        for i in range(16):
          ldi = ldv[i]
          row = q * 16 + i
          for k in range(D // 16):
            sl = pl.ds(k * 16, 16)
            acc[ldi, sl] = combine(acc[ldi, sl], gbuf[row, sl])
        return c2
      lax.fori_loop(0, GB // 16, sub_body, 0)

    ngrp = (cnt + (GB - 1)) // GB
    npairs = ngrp // 2
    @pl.when(ngrp > 0)
    def _():
      issue(0, gbufA, semA)
    @pl.when(ngrp > 1)
    def _():
      issue(1, gbufB, semB)

    def pair_body(p, carry):
      drain(gbufA, semA)
      process(2 * p, gbufA)
      @pl.when(2 * p + 2 < ngrp)
      def _():
        issue(2 * p + 2, gbufA, semA)
      drain(gbufB, semB)
      process(2 * p + 1, gbufB)
      @pl.when(2 * p + 3 < ngrp)
      def _():
        issue(2 * p + 3, gbufB, semB)
      return carry
    lax.fori_loop(0, npairs, pair_body, 0)

    @pl.when(ngrp > 2 * npairs)
    def _():
      drain(gbufA, semA)
      process(2 * npairs, gbufA)

    @pl.when(wid < NT - 1)
    def _():
      pltpu.sync_copy(acc.at[pl.ds(0, RPT)], out_hbm.at[pl.ds(lo, RPT)])

    @pl.when(wid == NT - 1)
    def _():
      pltpu.sync_copy(acc.at[pl.ds(0, LAST)], out_hbm.at[pl.ds(lo, LAST)])

  return sc_agg


_sc_max = _make_sc_agg("max")
_sc_add = _make_sc_agg("add")

BN = 1000


def _tc_layer1(x, Wrel1, Wroot1, b1):
  def body(x_ref, wr_ref, wro_ref, b_ref, xw_ref, xr_ref):
    xb = x_ref[...]
    for r in range(R):
      xw_ref[r] = jnp.dot(xb, wr_ref[r], preferred_element_type=jnp.float32)
    xr_ref[...] = (jnp.dot(xb, wro_ref[...], preferred_element_type=jnp.float32)
                   + b_ref[...])

  return pl.pallas_call(
      body,
      grid=(N // BN,),
      in_specs=[
          pl.BlockSpec((BN, D), lambda i: (i, 0)),
          pl.BlockSpec((R, D, D), lambda i: (0, 0, 0)),
          pl.BlockSpec((D, D), lambda i: (0, 0)),
          pl.BlockSpec((1, D), lambda i: (0, 0)),
      ],
      out_specs=[
          pl.BlockSpec((R, BN, D), lambda i: (0, i, 0)),
          pl.BlockSpec((BN, D), lambda i: (i, 0)),
      ],
      out_shape=[
          jax.ShapeDtypeStruct((R, N, D), jnp.float32),
          jax.ShapeDtypeStruct((N, D), jnp.float32),
      ],
  )(x, Wrel1, Wroot1, b1.reshape(1, D))


def _layer_norm_in(h, g, b):
  mu = jnp.mean(h, axis=1, keepdims=True)
  var = jnp.mean((h - mu) ** 2, axis=1, keepdims=True)
  return (h - mu) / jnp.sqrt(var + 1e-5) * g + b


def _tc_mid(agg1, xroot1, g1, be1, Wrel2, Wroot2, b2):
  def body(a_ref, xr_ref, g_ref, be_ref, wr_ref, wro_ref, b_ref,
           xw_ref, hr_ref):
    a = a_ref[...]
    a = jnp.where(a == NEG, 0.0, a)
    h = a + xr_ref[...]
    h = _layer_norm_in(h, g_ref[...], be_ref[...])
    h = jnp.maximum(h, 0.0)
    for r in range(R):
      xw_ref[r] = jnp.dot(h, wr_ref[r], preferred_element_type=jnp.float32)
    hr_ref[...] = (jnp.dot(h, wro_ref[...], preferred_element_type=jnp.float32)
                   + b_ref[...])

  return pl.pallas_call(
      body,
      grid=(N // BN,),
      in_specs=[
          pl.BlockSpec((BN, D), lambda i: (i, 0)),
          pl.BlockSpec((BN, D), lambda i: (i, 0)),
          pl.BlockSpec((1, D), lambda i: (0, 0)),
          pl.BlockSpec((1, D), lambda i: (0, 0)),
          pl.BlockSpec((R, D, D), lambda i: (0, 0, 0)),
          pl.BlockSpec((D, D), lambda i: (0, 0)),
          pl.BlockSpec((1, D), lambda i: (0, 0)),
      ],
      out_specs=[
          pl.BlockSpec((R, BN, D), lambda i: (0, i, 0)),
          pl.BlockSpec((BN, D), lambda i: (i, 0)),
      ],
      out_shape=[
          jax.ShapeDtypeStruct((R, N, D), jnp.float32),
          jax.ShapeDtypeStruct((N, D), jnp.float32),
      ],
  )(agg1, xroot1, g1.reshape(1, D), be1.reshape(1, D), Wrel2, Wroot2,
    b2.reshape(1, D))


def _tc_out(agg2, hroot2, g2, be2):
  def body(a_ref, hr_ref, g_ref, be_ref, o_ref):
    z = _layer_norm_in(a_ref[...] + hr_ref[...], g_ref[...], be_ref[...])
    z = z - jnp.max(z, axis=1, keepdims=True)
    o_ref[...] = z - jnp.log(jnp.sum(jnp.exp(z), axis=1, keepdims=True))

  return pl.pallas_call(
      body,
      grid=(N // BN,),
      in_specs=[
          pl.BlockSpec((BN, D), lambda i: (i, 0)),
          pl.BlockSpec((BN, D), lambda i: (i, 0)),
          pl.BlockSpec((1, D), lambda i: (0, 0)),
          pl.BlockSpec((1, D), lambda i: (0, 0)),
      ],
      out_specs=pl.BlockSpec((BN, D), lambda i: (i, 0)),
      out_shape=jax.ShapeDtypeStruct((N, D), jnp.float32),
  )(agg2, hroot2, g2.reshape(1, D), be2.reshape(1, D))


@jax.jit
def kernel(x, edge_index, edge_type, Wrel1, Wroot1, b1, g1, be1,
           Wrel2, Wroot2, b2, g2, be2):
  src = edge_index[0]
  dst = edge_index[1]
  rows = edge_type * N + src

  sel_rows, sel_ld, cnts = _sc_filter(rows, dst)
  xw1, xroot1 = _tc_layer1(x, Wrel1, Wroot1, b1)
  agg1 = _sc_max(xw1.reshape(R * N, D), sel_rows, sel_ld, cnts)
  xw2, hroot2 = _tc_mid(agg1, xroot1, g1, be1, Wrel2, Wroot2, b2)
  agg2 = _sc_add(xw2.reshape(R * N, D), sel_rows, sel_ld, cnts)
  return _tc_out(agg2, hroot2, g2, be2)
